# baseline (speedup 1.0000x reference)
import jax
import jax.numpy as jnp
from jax import lax
from jax.experimental import pallas as pl
from jax.experimental.pallas import tpu as pltpu
from jax.experimental.pallas import tpu_sc as plsc
from jax._src.pallas import core as pallas_core
from jax._src.pallas import mpmd
from jax._src.pallas.mosaic import core as tpu_core

N_ROWS = 1_000_000
N_COLS = 64
N_OUT = 16_384

ALIGNED = 999_936
COLS_PER_SC = N_COLS // 2
SEG = N_OUT // 16
NSUB = 16


def _scs_body(xt, xtail, idxt, ot, col_a, col_b, idx_v, out_v,
              dsem_a, dsem_b, rdy_a, rdy_b, done_a, done_b,
              isem_a, isem_b, gsem, osem_a, osem_b):
    cid = lax.axis_index("c")
    j0 = cid * COLS_PER_SC

    def stage(col_ref, j, dsem, do_wait):
        slc = ALIGNED // 12
        copies = [
            (xt.at[j, pl.ds(s * slc, slc)], col_ref.at[pl.ds(s * slc, slc)])
            for s in range(12)
        ] + [
            (xtail.at[pl.ds(j * 128, 128)], col_ref.at[pl.ds(ALIGNED, 128)]),
        ]
        for s, d in copies:
            cp = pltpu.make_async_copy(s, d, dsem)
            cp.wait() if do_wait else cp.start()

    stage(col_a, j0, dsem_a, False)
    for k in range(COLS_PER_SC):
        buf, dsem = (col_a, dsem_a) if k % 2 == 0 else (col_b, dsem_b)
        rdy = rdy_a if k % 2 == 0 else rdy_b
        stage(buf, j0 + k, dsem, True)
        for t in range(NSUB):
            pltpu.semaphore_signal(rdy, 1, device_id={"s": t})
        if k + 1 < COLS_PER_SC:
            nbuf, ndsem = (col_b, dsem_b) if k % 2 == 0 else (col_a, dsem_a)
            if k >= 1:
                pltpu.semaphore_wait(done_b if k % 2 == 0 else done_a, NSUB)
            stage(nbuf, j0 + k + 1, ndsem, False)


def _tec_body(xt, xtail, idxt, ot, col_a, col_b, idx_v, out_v,
              dsem_a, dsem_b, rdy_a, rdy_b, done_a, done_b,
              isem_a, isem_b, gsem, osem_a, osem_b):
    cid = lax.axis_index("c")
    sid = lax.axis_index("s")
    j0 = cid * COLS_PER_SC

    def idx_slot(k):
        return idx_v.at[pl.ds((k % 2) * SEG, SEG)]

    def out_slot(k):
        return out_v.at[pl.ds((k % 2) * SEG, SEG)]

    def idx_copy(k, do_wait):
        cp = pltpu.make_async_copy(idxt.at[j0 + k, pl.ds(sid * SEG, SEG)],
                                   idx_slot(k), isem_a if k % 2 == 0 else isem_b)
        cp.wait() if do_wait else cp.start()

    def out_copy(k, do_wait):
        cp = pltpu.make_async_copy(out_slot(k), ot.at[j0 + k, pl.ds(sid * SEG, SEG)],
                                   osem_a if k % 2 == 0 else osem_b)
        cp.wait() if do_wait else cp.start()

    idx_copy(0, False)
    for k in range(COLS_PER_SC):
        buf = col_a if k % 2 == 0 else col_b
        rdy = rdy_a if k % 2 == 0 else rdy_b
        done = done_a if k % 2 == 0 else done_b
        if k + 1 < COLS_PER_SC:
            idx_copy(k + 1, False)
        idx_copy(k, True)
        if k >= 2:
            out_copy(k - 2, True)
        pl.semaphore_wait(rdy, 1)

        pltpu.make_async_copy(buf.at[idx_slot(k)], out_slot(k), gsem).start()
        pltpu.make_async_copy(buf.at[idx_slot(k)], out_slot(k), gsem).wait()
        pltpu.semaphore_signal(done, 1)
        out_copy(k, False)

    out_copy(COLS_PER_SC - 2, True)
    out_copy(COLS_PER_SC - 1, True)


@jax.jit
def _gather_sc(xt, xtail, idxt):
    smesh = plsc.ScalarSubcoreMesh(axis_name="c", num_cores=2)
    vmesh = plsc.VectorSubcoreMesh(core_axis_name="c", subcore_axis_name="s")
    sem = tpu_core.MemorySpace.SEMAPHORE
    dma_aval = pltpu.SemaphoreType.DMA(()).inner_aval
    reg_aval = pltpu.SemaphoreType.REGULAR(()).inner_aval

    def scs_sem(aval):
        return pallas_core.MemoryRef(aval, pallas_core.CoreMemorySpace(sem, smesh))

    def tec_sem(aval):
        return pallas_core.MemoryRef(aval, pallas_core.CoreMemorySpace(sem, vmesh))

    tec_vmem = pallas_core.CoreMemorySpace(tpu_core.MemorySpace.VMEM, vmesh)

    return mpmd.mpmd_map(
        [(smesh, _scs_body), (vmesh, _tec_body)],
        out_types=jax.ShapeDtypeStruct((N_COLS, N_OUT), jnp.float32),
        scratch_types=[
            pltpu.VMEM_SHARED((ALIGNED + 128,), jnp.float32),
            pltpu.VMEM_SHARED((ALIGNED + 128,), jnp.float32),
            tec_vmem((2 * SEG,), jnp.int32),
            tec_vmem((2 * SEG,), jnp.float32),
            scs_sem(dma_aval),
            scs_sem(dma_aval),
            tec_sem(reg_aval),
            tec_sem(reg_aval),
            scs_sem(reg_aval),
            scs_sem(reg_aval),
            tec_sem(dma_aval),
            tec_sem(dma_aval),
            tec_sem(dma_aval),
            tec_sem(dma_aval),
            tec_sem(dma_aval),
        ],
    )(xt, xtail, idxt)


def kernel(x, dim, index, sparse_grad, out):
    xtail = jnp.pad(x[ALIGNED:, :], ((0, 128 - (N_ROWS - ALIGNED)), (0, 0)))
    res_t = _gather_sc(x.T, xtail.T.reshape(-1), index.astype(jnp.int32).T)
    return res_t.T


# speedup vs baseline: 1.1605x; 1.1605x over previous
import jax
import jax.numpy as jnp
from jax import lax
from jax.experimental import pallas as pl
from jax.experimental.pallas import tpu as pltpu
from jax.experimental.pallas import tpu_sc as plsc

N_ROWS = 1_000_000
N_COLS = 64
N_OUT = 16_384

ALIGNED = 999_936
COLS_PER_SC = N_COLS // 2
SEG = N_OUT // 16
SLC = 488 * 128
SLC_LAST = ALIGNED - 15 * SLC


def _gather_body(xt, xtail, idxt, ot, col_a, col_b, idx_v, out_v,
                 sem_a, sem_b, isem_a, isem_b, gsem, osem_a, osem_b):
    cid = lax.axis_index("c")
    sid = lax.axis_index("s")
    j0 = cid * COLS_PER_SC

    def stage_halves(col_ref, j):
        h = SLC // 2
        parts = [(sid * SLC, h), (sid * SLC + h, SLC - h)]
        return [(xt.at[j, pl.ds(o, n)], col_ref.at[pl.ds(o, n)]) for o, n in parts]

    def stage_last(col_ref, j):
        h = 246 * 128
        return [(xt.at[j, pl.ds(15 * SLC, h)], col_ref.at[pl.ds(15 * SLC, h)]),
                (xt.at[j, pl.ds(15 * SLC + h, SLC_LAST - h)],
                 col_ref.at[pl.ds(15 * SLC + h, SLC_LAST - h)]),
                (xtail.at[pl.ds(j * 128, 128)], col_ref.at[pl.ds(ALIGNED, 128)])]

    def stage_start(col_ref, j, sem):
        @pl.when(sid < 15)
        def _():
            for s, d in stage_halves(col_ref, j):
                pltpu.make_async_copy(s, d, sem).start()

        @pl.when(sid == 15)
        def _():
            for s, d in stage_last(col_ref, j):
                pltpu.make_async_copy(s, d, sem).start()

    def stage_wait(col_ref, j, sem):
        @pl.when(sid < 15)
        def _():
            for s, d in stage_halves(col_ref, j):
                pltpu.make_async_copy(s, d, sem).wait()

        @pl.when(sid == 15)
        def _():
            for s, d in stage_last(col_ref, j):
                pltpu.make_async_copy(s, d, sem).wait()

    def idx_slot(k):
        return idx_v.at[pl.ds((k % 2) * SEG, SEG)]

    def out_slot(k):
        return out_v.at[pl.ds((k % 2) * SEG, SEG)]

    def idx_start(k):
        pltpu.make_async_copy(idxt.at[j0 + k, pl.ds(sid * SEG, SEG)],
                              idx_slot(k), isem_a if k % 2 == 0 else isem_b).start()

    def idx_wait(k):
        pltpu.make_async_copy(idxt.at[j0 + k, pl.ds(sid * SEG, SEG)],
                              idx_slot(k), isem_a if k % 2 == 0 else isem_b).wait()

    def out_start(k):
        pltpu.make_async_copy(out_slot(k), ot.at[j0 + k, pl.ds(sid * SEG, SEG)],
                              osem_a if k % 2 == 0 else osem_b).start()

    def out_wait(k):
        pltpu.make_async_copy(out_slot(k), ot.at[j0 + k, pl.ds(sid * SEG, SEG)],
                              osem_a if k % 2 == 0 else osem_b).wait()

    stage_start(col_a, j0, sem_a)
    idx_start(0)

    for k in range(COLS_PER_SC):
        j = j0 + k
        buf, sem = (col_a, sem_a) if k % 2 == 0 else (col_b, sem_b)
        if k + 1 < COLS_PER_SC:
            nbuf, nsem = (col_b, sem_b) if k % 2 == 0 else (col_a, sem_a)
            stage_start(nbuf, j + 1, nsem)
            idx_start(k + 1)
        stage_wait(buf, j, sem)
        idx_wait(k)
        if k >= 2:
            out_wait(k - 2)
        plsc.subcore_barrier()

        pltpu.make_async_copy(buf.at[idx_slot(k)], out_slot(k), gsem).start()
        pltpu.make_async_copy(buf.at[idx_slot(k)], out_slot(k), gsem).wait()
        out_start(k)
        plsc.subcore_barrier()

    out_wait(COLS_PER_SC - 2)
    out_wait(COLS_PER_SC - 1)


@jax.jit
def _gather_sc(xt, xtail, idxt):
    mesh = plsc.VectorSubcoreMesh(core_axis_name="c", subcore_axis_name="s")
    return pl.kernel(
        _gather_body,
        out_type=jax.ShapeDtypeStruct((N_COLS, N_OUT), jnp.float32),
        mesh=mesh,
        scratch_types=[
            pltpu.VMEM_SHARED((ALIGNED + 128,), jnp.float32),
            pltpu.VMEM_SHARED((ALIGNED + 128,), jnp.float32),
            pltpu.VMEM((2 * SEG,), jnp.int32),
            pltpu.VMEM((2 * SEG,), jnp.float32),
            pltpu.SemaphoreType.DMA,
            pltpu.SemaphoreType.DMA,
            pltpu.SemaphoreType.DMA,
            pltpu.SemaphoreType.DMA,
            pltpu.SemaphoreType.DMA,
            pltpu.SemaphoreType.DMA,
            pltpu.SemaphoreType.DMA,
        ],
    )(xt, xtail, idxt)


def kernel(x, dim, index, sparse_grad, out):
    xtail = jnp.pad(x[ALIGNED:, :], ((0, 128 - (N_ROWS - ALIGNED)), (0, 0)))
    res_t = _gather_sc(x.T, xtail.T.reshape(-1), index.astype(jnp.int32).T)
    return res_t.T


# speedup vs baseline: 1.1895x; 1.0250x over previous
import jax
import jax.numpy as jnp
from jax import lax
from jax.experimental import pallas as pl
from jax.experimental.pallas import tpu as pltpu
from jax.experimental.pallas import tpu_sc as plsc
from jax._src.pallas import core as pallas_core
from jax._src.pallas import mpmd
from jax._src.pallas.mosaic import core as tpu_core

N_ROWS = 1_000_000
N_COLS = 64
N_OUT = 16_384

ALIGNED = 999_936
COLS_PER_SC = N_COLS // 2
SEG = N_OUT // 16
NSUB = 16

TSLC = 279 * 128
SCS_OFF = NSUB * TSLC
SCS_LEN = ALIGNED - SCS_OFF
SCS_NSLC = 6
SCS_SLC = SCS_LEN // SCS_NSLC


def _scs_body(xt, xtail, idxt, ot, col_a, col_b, idx_v, out_v,
              dsem_a, dsem_b, rdy_a, rdy_b, done_a, done_b,
              ssem_a, ssem_b, isem_a, isem_b, gsem, osem_a, osem_b):
    cid = lax.axis_index("c")
    j0 = cid * COLS_PER_SC

    def stage(col_ref, j, dsem, do_wait):
        copies = [
            (xt.at[j, pl.ds(SCS_OFF + s * SCS_SLC, SCS_SLC)],
             col_ref.at[pl.ds(SCS_OFF + s * SCS_SLC, SCS_SLC)])
            for s in range(SCS_NSLC)
        ] + [
            (xtail.at[pl.ds(j * 128, 128)], col_ref.at[pl.ds(ALIGNED, 128)]),
        ]
        for s, d in copies:
            cp = pltpu.make_async_copy(s, d, dsem)
            cp.wait() if do_wait else cp.start()

    stage(col_a, j0, dsem_a, False)
    for k in range(COLS_PER_SC):
        buf, dsem = (col_a, dsem_a) if k % 2 == 0 else (col_b, dsem_b)
        rdy = rdy_a if k % 2 == 0 else rdy_b
        stage(buf, j0 + k, dsem, True)
        for t in range(NSUB):
            pltpu.semaphore_signal(rdy, 1, device_id={"s": t})
        if k + 1 < COLS_PER_SC:
            nbuf, ndsem = (col_b, dsem_b) if k % 2 == 0 else (col_a, dsem_a)
            if k >= 1:
                pltpu.semaphore_wait(done_b if k % 2 == 0 else done_a, NSUB)
            stage(nbuf, j0 + k + 1, ndsem, False)


def _tec_body(xt, xtail, idxt, ot, col_a, col_b, idx_v, out_v,
              dsem_a, dsem_b, rdy_a, rdy_b, done_a, done_b,
              ssem_a, ssem_b, isem_a, isem_b, gsem, osem_a, osem_b):
    cid = lax.axis_index("c")
    sid = lax.axis_index("s")
    j0 = cid * COLS_PER_SC

    def stage_copy(col_ref, j, ssem, do_wait):
        cp = pltpu.make_async_copy(xt.at[j, pl.ds(sid * TSLC, TSLC)],
                                   col_ref.at[pl.ds(sid * TSLC, TSLC)], ssem)
        cp.wait() if do_wait else cp.start()

    def idx_slot(k):
        return idx_v.at[pl.ds((k % 2) * SEG, SEG)]

    def out_slot(k):
        return out_v.at[pl.ds((k % 2) * SEG, SEG)]

    def idx_copy(k, do_wait):
        cp = pltpu.make_async_copy(idxt.at[j0 + k, pl.ds(sid * SEG, SEG)],
                                   idx_slot(k), isem_a if k % 2 == 0 else isem_b)
        cp.wait() if do_wait else cp.start()

    def out_copy(k, do_wait):
        cp = pltpu.make_async_copy(out_slot(k), ot.at[j0 + k, pl.ds(sid * SEG, SEG)],
                                   osem_a if k % 2 == 0 else osem_b)
        cp.wait() if do_wait else cp.start()

    stage_copy(col_a, j0, ssem_a, False)
    idx_copy(0, False)
    for k in range(COLS_PER_SC):
        j = j0 + k
        buf, ssem = (col_a, ssem_a) if k % 2 == 0 else (col_b, ssem_b)
        rdy = rdy_a if k % 2 == 0 else rdy_b
        done = done_a if k % 2 == 0 else done_b
        if k + 1 < COLS_PER_SC:
            nbuf, nssem = (col_b, ssem_b) if k % 2 == 0 else (col_a, ssem_a)
            stage_copy(nbuf, j + 1, nssem, False)
            idx_copy(k + 1, False)
        stage_copy(buf, j, ssem, True)
        idx_copy(k, True)
        if k >= 2:
            out_copy(k - 2, True)
        pl.semaphore_wait(rdy, 1)
        plsc.subcore_barrier()

        pltpu.make_async_copy(buf.at[idx_slot(k)], out_slot(k), gsem).start()
        pltpu.make_async_copy(buf.at[idx_slot(k)], out_slot(k), gsem).wait()
        pltpu.semaphore_signal(done, 1)
        out_copy(k, False)
        plsc.subcore_barrier()

    out_copy(COLS_PER_SC - 2, True)
    out_copy(COLS_PER_SC - 1, True)


@jax.jit
def _gather_sc(xt, xtail, idxt):
    smesh = plsc.ScalarSubcoreMesh(axis_name="c", num_cores=2)
    vmesh = plsc.VectorSubcoreMesh(core_axis_name="c", subcore_axis_name="s")
    sem_space = tpu_core.MemorySpace.SEMAPHORE
    dma_aval = pltpu.SemaphoreType.DMA(()).inner_aval
    reg_aval = pltpu.SemaphoreType.REGULAR(()).inner_aval

    def scs_sem(aval):
        return pallas_core.MemoryRef(aval, pallas_core.CoreMemorySpace(sem_space, smesh))

    def tec_sem(aval):
        return pallas_core.MemoryRef(aval, pallas_core.CoreMemorySpace(sem_space, vmesh))

    tec_vmem = pallas_core.CoreMemorySpace(tpu_core.MemorySpace.VMEM, vmesh)

    return mpmd.mpmd_map(
        [(smesh, _scs_body), (vmesh, _tec_body)],
        out_types=jax.ShapeDtypeStruct((N_COLS, N_OUT), jnp.float32),
        scratch_types=[
            pltpu.VMEM_SHARED((ALIGNED + 128,), jnp.float32),
            pltpu.VMEM_SHARED((ALIGNED + 128,), jnp.float32),
            tec_vmem((2 * SEG,), jnp.int32),
            tec_vmem((2 * SEG,), jnp.float32),
            scs_sem(dma_aval),
            scs_sem(dma_aval),
            tec_sem(reg_aval),
            tec_sem(reg_aval),
            scs_sem(reg_aval),
            scs_sem(reg_aval),
            tec_sem(dma_aval),
            tec_sem(dma_aval),
            tec_sem(dma_aval),
            tec_sem(dma_aval),
            tec_sem(dma_aval),
            tec_sem(dma_aval),
            tec_sem(dma_aval),
        ],
    )(xt, xtail, idxt)


def kernel(x, dim, index, sparse_grad, out):
    xtail = jnp.pad(x[ALIGNED:, :], ((0, 128 - (N_ROWS - ALIGNED)), (0, 0)))
    res_t = _gather_sc(x.T, xtail.T.reshape(-1), index.astype(jnp.int32).T)
    return res_t.T
